# baseline (speedup 1.0000x reference)
import functools

import jax
import jax.numpy as jnp
from jax import lax
from jax.experimental import pallas as pl
from jax.experimental.pallas import tpu as pltpu
from jax.experimental.pallas import tpu_sc as plsc

U = 25000
N = 50000
D = 64
NNZ = 800000
B = 16384

HALF = 25000
ACC_R = 25152
DUM_BASE = 25000
EPT = NNZ // 16
SB = 1024
BLK = 128
NSB = 48
TAIL = EPT - NSB * SB

_mesh = plsc.VectorSubcoreMesh(
    core_axis_name="c", subcore_axis_name="s", num_cores=2, num_subcores=16)

def _layer_body(ego, rows, cols, vals, out, colv, rowv, valv, dstv, gbuf,
                sbuf, zbuf, acc, sem):
    c = lax.axis_index("c")
    s = lax.axis_index("s")
    base = c * HALF
    lane = lax.iota(jnp.int32, 16)
    dum = DUM_BASE + s * 8 + (lane & 7)
    _Z16 = jnp.zeros((16,), jnp.float32)
    _Z16I = jnp.zeros((16,), jnp.int32)

    def _zrow(r, _):
        for g in range(4):
            zbuf[r, pl.ds(g * 16, 16)] = _Z16
        return 0
    lax.fori_loop(0, 12, _zrow, 0)

    def _zcopy(k, _):
        pltpu.sync_copy(zbuf, acc.at[pl.ds(s * 1572 + k * 12, 12)])
        return 0
    lax.fori_loop(0, 131, _zcopy, 0)
    plsc.subcore_barrier()

    estart = s * EPT

    def _dst_group(g, _):
        rv = rowv[pl.ds(g * 16, 16)]
        local = rv - base
        msk = (local >= 0) & (local < HALF)
        dv = jnp.where(msk, local, dum)
        j = g // 8
        k = g - j * 8
        dstv[j, pl.ds(k * 16, 16)] = dv
        return 0

    def _process_superblock():
        lax.fori_loop(0, SB // 16, _dst_group, 0)

        def _pair(q, _):
            descs = [
                pltpu.async_copy(
                    ego.at[colv.at[pl.ds((q * 2 + u) * BLK, BLK)]],
                    gbuf.at[u], sem)
                for u in range(2)
            ]
            for d in descs:
                d.wait()

            def _chunk(u, _):
                j = q * 2 + u
                for g in range(BLK // 16):
                    vv = valv[pl.ds(j * BLK + g * 16, 16)]
                    for l in range(16):
                        v = vv[l]
                        e = g * 16 + l
                        for cg in range(4):
                            sl = pl.ds(cg * 16, 16)
                            sbuf[e, sl] = gbuf[u, e, sl] * v
                pltpu.sync_copy(sbuf, acc.at[dstv.at[j]], add=True)
                return 0
            lax.fori_loop(0, 2, _chunk, 0)
            return 0
        lax.fori_loop(0, SB // (2 * BLK), _pair, 0)

    def _full_sb(b, _):
        off = estart + b * SB
        pltpu.sync_copy(cols.at[pl.ds(off, SB)], colv)
        pltpu.sync_copy(vals.at[pl.ds(off, SB)], valv)
        pltpu.sync_copy(rows.at[pl.ds(off, SB)], rowv)
        _process_superblock()
        return 0
    lax.fori_loop(0, NSB, _full_sb, 0)

    toff = estart + NSB * SB
    pltpu.sync_copy(cols.at[pl.ds(toff, TAIL)], colv.at[pl.ds(0, TAIL)])
    pltpu.sync_copy(vals.at[pl.ds(toff, TAIL)], valv.at[pl.ds(0, TAIL)])
    pltpu.sync_copy(rows.at[pl.ds(toff, TAIL)], rowv.at[pl.ds(0, TAIL)])
    for p in range(TAIL // 16, SB // 16):
        colv[pl.ds(p * 16, 16)] = _Z16I
        valv[pl.ds(p * 16, 16)] = _Z16
        rowv[pl.ds(p * 16, 16)] = _Z16I - 1
    _process_superblock()

    plsc.subcore_barrier()
    pltpu.sync_copy(acc.at[pl.ds(s * 1560, 1560)],
                    out.at[pl.ds(base + s * 1560, 1560)])
    @pl.when(s == 15)
    def _():
        pltpu.sync_copy(acc.at[pl.ds(24960, 40)],
                        out.at[pl.ds(base + 24960, 40)])


_layer = pl.kernel(
    _layer_body,
    out_type=jax.ShapeDtypeStruct((N, D), jnp.float32),
    mesh=_mesh,
    scratch_types=[
        pltpu.VMEM((SB,), jnp.int32),
        pltpu.VMEM((SB,), jnp.int32),
        pltpu.VMEM((SB,), jnp.float32),
        pltpu.VMEM((8, BLK), jnp.int32),
        pltpu.VMEM((2, BLK, D), jnp.float32),
        pltpu.VMEM((BLK, D), jnp.float32),
        pltpu.VMEM((12, D), jnp.float32),
        pltpu.VMEM_SHARED((ACC_R, D), jnp.float32),
        pltpu.SemaphoreType.DMA,
    ],
    compiler_params=pltpu.CompilerParams(use_tc_tiling_on_sc=False),
)


def _final_body(e0, e1, e2, e3, sel, out, idxv, g0, g1, g2, g3, obuf, sem):
    c = lax.axis_index("c")
    s = lax.axis_index("s")
    wid = s * 2 + c
    rpw = (2 * B) // 32

    def _sum_row(e, _):
        for g in range(4):
            sl = pl.ds(g * 16, 16)
            obuf[e, sl] = (g0[e, sl] + g1[e, sl] + g2[e, sl] + g3[e, sl]) * 0.25
        return 0

    def _block(blk, _):
        off = wid * rpw + blk * BLK
        pltpu.sync_copy(sel.at[pl.ds(off, BLK)], idxv)
        pltpu.async_copy(e0.at[idxv], g0, sem).wait()
        pltpu.async_copy(e1.at[idxv], g1, sem).wait()
        pltpu.async_copy(e2.at[idxv], g2, sem).wait()
        pltpu.async_copy(e3.at[idxv], g3, sem).wait()
        lax.fori_loop(0, BLK, _sum_row, 0)
        pltpu.sync_copy(obuf, out.at[pl.ds(off, BLK)])
        return 0
    lax.fori_loop(0, rpw // BLK, _block, 0)


_finalize = pl.kernel(
    _final_body,
    out_type=jax.ShapeDtypeStruct((2 * B, D), jnp.float32),
    mesh=_mesh,
    scratch_types=[
        pltpu.VMEM((BLK,), jnp.int32),
        pltpu.VMEM((BLK, D), jnp.float32),
        pltpu.VMEM((BLK, D), jnp.float32),
        pltpu.VMEM((BLK, D), jnp.float32),
        pltpu.VMEM((BLK, D), jnp.float32),
        pltpu.VMEM((BLK, D), jnp.float32),
        pltpu.SemaphoreType.DMA,
    ],
    compiler_params=pltpu.CompilerParams(use_tc_tiling_on_sc=False),
)


def _pred_body(x_ref, wt_ref, b_ref, o_ref):
    o_ref[...] = jnp.dot(x_ref[...], wt_ref[...],
                         preferred_element_type=jnp.float32) + b_ref[...]


_predict = pl.pallas_call(
    _pred_body,
    out_shape=jax.ShapeDtypeStruct((2 * B, D), jnp.float32),
    grid=(8,),
    in_specs=[
        pl.BlockSpec((2 * B // 8, D), lambda i: (i, 0)),
        pl.BlockSpec((D, D), lambda i: (0, 0)),
        pl.BlockSpec((1, D), lambda i: (0, 0)),
    ],
    out_specs=pl.BlockSpec((2 * B // 8, D), lambda i: (i, 0)),
)


def kernel(user, item, adj_row, adj_col, adj_val, user_emb, item_emb, W, b):
    ego0 = jnp.concatenate([user_emb, item_emb], axis=0)
    sel = jnp.concatenate([user, item + U], axis=0)
    e1 = _layer(ego0, adj_row, adj_col, adj_val)
    e2 = _layer(e1, adj_row, adj_col, adj_val)
    e3 = _layer(e2, adj_row, adj_col, adj_val)
    sm = _finalize(ego0, e1, e2, e3, sel)
    pred = _predict(sm, W.T, b.reshape(1, D))
    return (pred[:B], sm[:B], pred[B:], sm[B:])


# speedup vs baseline: 2.0172x; 2.0172x over previous
import functools

import jax
import jax.numpy as jnp
from jax import lax
from jax.experimental import pallas as pl
from jax.experimental.pallas import tpu as pltpu
from jax.experimental.pallas import tpu_sc as plsc

U = 25000
N = 50000
D = 64
NNZ = 800000
B = 16384

HALF = 25000
ACC_R = 25152
DUM_BASE = 25000
EPT = NNZ // 16
SB = 1024
BLK = 128
NSB = 48
TAIL = EPT - NSB * SB

_mesh = plsc.VectorSubcoreMesh(
    core_axis_name="c", subcore_axis_name="s", num_cores=2, num_subcores=16)

def _layer_body(ego, rows, cols, vals, out, colv, rowv, valv, dstv, gbuf,
                sbuf, zbuf, acc, sem):
    c = lax.axis_index("c")
    s = lax.axis_index("s")
    base = c * HALF
    lane = lax.iota(jnp.int32, 16)
    dum = DUM_BASE + s * 8 + (lane & 7)
    _Z16 = jnp.zeros((16,), jnp.float32)
    _Z16I = jnp.zeros((16,), jnp.int32)

    def _zrow(r, _):
        for g in range(4):
            zbuf[r, pl.ds(g * 16, 16)] = _Z16
        return 0
    lax.fori_loop(0, 12, _zrow, 0)

    def _zcopy(k, _):
        pltpu.sync_copy(zbuf, acc.at[pl.ds(s * 1572 + k * 12, 12)])
        return 0
    lax.fori_loop(0, 131, _zcopy, 0)
    plsc.subcore_barrier()

    estart = s * EPT

    def _dst_group(g, _):
        rv = rowv[pl.ds(g * 16, 16)]
        local = rv - base
        msk = (local >= 0) & (local < HALF)
        dv = jnp.where(msk, local, dum)
        j = g // 8
        k = g - j * 8
        dstv[j, pl.ds(k * 16, 16)] = dv
        return 0

    def _process_superblock():
        lax.fori_loop(0, SB // 16, _dst_group, 0)

        def _sub_block(j, _):
            pltpu.async_copy(
                ego.at[colv.at[pl.ds(j * BLK, BLK)]], gbuf, sem).wait()
            for g in range(BLK // 16):
                vv = valv[pl.ds(j * BLK + g * 16, 16)]
                for l in range(16):
                    v = vv[l]
                    e = g * 16 + l
                    for cg in range(4):
                        sl = pl.ds(cg * 16, 16)
                        sbuf[e, sl] = gbuf[e, sl] * v
            pltpu.sync_copy(sbuf, acc.at[dstv.at[j]], add=True)
            return 0
        lax.fori_loop(0, SB // BLK, _sub_block, 0)

    def _full_sb(b, _):
        off = estart + b * SB
        pltpu.sync_copy(cols.at[pl.ds(off, SB)], colv)
        pltpu.sync_copy(vals.at[pl.ds(off, SB)], valv)
        pltpu.sync_copy(rows.at[pl.ds(off, SB)], rowv)
        _process_superblock()
        return 0
    lax.fori_loop(0, NSB, _full_sb, 0)

    toff = estart + NSB * SB
    pltpu.sync_copy(cols.at[pl.ds(toff, TAIL)], colv.at[pl.ds(0, TAIL)])
    pltpu.sync_copy(vals.at[pl.ds(toff, TAIL)], valv.at[pl.ds(0, TAIL)])
    pltpu.sync_copy(rows.at[pl.ds(toff, TAIL)], rowv.at[pl.ds(0, TAIL)])
    for p in range(TAIL // 16, SB // 16):
        colv[pl.ds(p * 16, 16)] = _Z16I
        valv[pl.ds(p * 16, 16)] = _Z16
        rowv[pl.ds(p * 16, 16)] = _Z16I - 1
    _process_superblock()

    plsc.subcore_barrier()
    pltpu.sync_copy(acc.at[pl.ds(s * 1560, 1560)],
                    out.at[pl.ds(base + s * 1560, 1560)])
    @pl.when(s == 15)
    def _():
        pltpu.sync_copy(acc.at[pl.ds(24960, 40)],
                        out.at[pl.ds(base + 24960, 40)])


_layer = pl.kernel(
    _layer_body,
    out_type=jax.ShapeDtypeStruct((N, D), jnp.float32),
    mesh=_mesh,
    scratch_types=[
        pltpu.VMEM((SB,), jnp.int32),
        pltpu.VMEM((SB,), jnp.int32),
        pltpu.VMEM((SB,), jnp.float32),
        pltpu.VMEM((8, BLK), jnp.int32),
        pltpu.VMEM((BLK, D), jnp.float32),
        pltpu.VMEM((BLK, D), jnp.float32),
        pltpu.VMEM((12, D), jnp.float32),
        pltpu.VMEM_SHARED((ACC_R, D), jnp.float32),
        pltpu.SemaphoreType.DMA,
    ],
    compiler_params=pltpu.CompilerParams(use_tc_tiling_on_sc=False),
)


def _final_body(e0, e1, e2, e3, sel, out, idxv, g0, g1, g2, g3, obuf, sem):
    c = lax.axis_index("c")
    s = lax.axis_index("s")
    wid = s * 2 + c
    rpw = (2 * B) // 32

    def _sum_row(e, _):
        for g in range(4):
            sl = pl.ds(g * 16, 16)
            obuf[e, sl] = (g0[e, sl] + g1[e, sl] + g2[e, sl] + g3[e, sl]) * 0.25
        return 0

    def _block(blk, _):
        off = wid * rpw + blk * BLK
        pltpu.sync_copy(sel.at[pl.ds(off, BLK)], idxv)
        pltpu.async_copy(e0.at[idxv], g0, sem).wait()
        pltpu.async_copy(e1.at[idxv], g1, sem).wait()
        pltpu.async_copy(e2.at[idxv], g2, sem).wait()
        pltpu.async_copy(e3.at[idxv], g3, sem).wait()
        lax.fori_loop(0, BLK, _sum_row, 0)
        pltpu.sync_copy(obuf, out.at[pl.ds(off, BLK)])
        return 0
    lax.fori_loop(0, rpw // BLK, _block, 0)


_finalize = pl.kernel(
    _final_body,
    out_type=jax.ShapeDtypeStruct((2 * B, D), jnp.float32),
    mesh=_mesh,
    scratch_types=[
        pltpu.VMEM((BLK,), jnp.int32),
        pltpu.VMEM((BLK, D), jnp.float32),
        pltpu.VMEM((BLK, D), jnp.float32),
        pltpu.VMEM((BLK, D), jnp.float32),
        pltpu.VMEM((BLK, D), jnp.float32),
        pltpu.VMEM((BLK, D), jnp.float32),
        pltpu.SemaphoreType.DMA,
    ],
    compiler_params=pltpu.CompilerParams(use_tc_tiling_on_sc=False),
)


def _pred_body(x_ref, wt_ref, b_ref, o_ref):
    o_ref[...] = jnp.dot(x_ref[...], wt_ref[...],
                         preferred_element_type=jnp.float32) + b_ref[...]


_predict = pl.pallas_call(
    _pred_body,
    out_shape=jax.ShapeDtypeStruct((2 * B, D), jnp.float32),
    grid=(8,),
    in_specs=[
        pl.BlockSpec((2 * B // 8, D), lambda i: (i, 0)),
        pl.BlockSpec((D, D), lambda i: (0, 0)),
        pl.BlockSpec((1, D), lambda i: (0, 0)),
    ],
    out_specs=pl.BlockSpec((2 * B // 8, D), lambda i: (i, 0)),
)


def kernel(user, item, adj_row, adj_col, adj_val, user_emb, item_emb, W, b):
    ego0 = jnp.concatenate([user_emb, item_emb], axis=0)
    sel = jnp.concatenate([user, item + U], axis=0)
    e1 = _layer(ego0, adj_row, adj_col, adj_val)
    e2 = _layer(e1, adj_row, adj_col, adj_val)
    e3 = _layer(e2, adj_row, adj_col, adj_val)
    sm = _finalize(ego0, e1, e2, e3, sel)
    pred = _predict(sm, W.T, b.reshape(1, D))
    return (pred[:B], sm[:B], pred[B:], sm[B:])


# speedup vs baseline: 2.3355x; 1.1578x over previous
import functools

import jax
import jax.numpy as jnp
from jax import lax
from jax.experimental import pallas as pl
from jax.experimental.pallas import tpu as pltpu
from jax.experimental.pallas import tpu_sc as plsc

U = 25000
N = 50000
D = 64
NNZ = 800000
B = 16384

HALF = 25000
ACC_R = 25152
DUM_BASE = 25000
EPT = NNZ // 16
SB = 1024
BLK = 128
NSB = 48
TAIL = EPT - NSB * SB

_mesh = plsc.VectorSubcoreMesh(
    core_axis_name="c", subcore_axis_name="s", num_cores=2, num_subcores=16)

def _layer_body(ego, rows, cols, vals, out, colv, rowv, valv, dstv, gbufa,
                gbufb, sbuf, zbuf, acc, sema, semb):
    c = lax.axis_index("c")
    s = lax.axis_index("s")
    base = c * HALF
    lane = lax.iota(jnp.int32, 16)
    dum = DUM_BASE + s * 8 + (lane & 7)
    _Z16 = jnp.zeros((16,), jnp.float32)
    _Z16I = jnp.zeros((16,), jnp.int32)

    def _zrow(r, _):
        for g in range(4):
            zbuf[r, pl.ds(g * 16, 16)] = _Z16
        return 0
    lax.fori_loop(0, 12, _zrow, 0)

    def _zcopy(k, _):
        pltpu.sync_copy(zbuf, acc.at[pl.ds(s * 1572 + k * 12, 12)])
        return 0
    lax.fori_loop(0, 131, _zcopy, 0)
    plsc.subcore_barrier()

    estart = s * EPT

    def _dst_group(g, _):
        rv = rowv[pl.ds(g * 16, 16)]
        local = rv - base
        msk = (local >= 0) & (local < HALF)
        dv = jnp.where(msk, local, dum)
        j = g // 8
        k = g - j * 8
        dstv[j, pl.ds(k * 16, 16)] = dv
        return 0

    def _process_superblock():
        lax.fori_loop(0, SB // 16, _dst_group, 0)

        def _scale_scatter(j, gb):
            for g in range(BLK // 16):
                vv = valv[pl.ds(j * BLK + g * 16, 16)]
                for l in range(16):
                    v = vv[l]
                    e = g * 16 + l
                    for cg in range(4):
                        sl = pl.ds(cg * 16, 16)
                        sbuf[e, sl] = gb[e, sl] * v
            pltpu.sync_copy(sbuf, acc.at[dstv.at[j]], add=True)

        def _pair(q, _):
            ja = q * 2
            jb = q * 2 + 1
            da = pltpu.async_copy(
                ego.at[colv.at[pl.ds(ja * BLK, BLK)]], gbufa, sema)
            db = pltpu.async_copy(
                ego.at[colv.at[pl.ds(jb * BLK, BLK)]], gbufb, semb)
            da.wait()
            _scale_scatter(ja, gbufa)
            db.wait()
            _scale_scatter(jb, gbufb)
            return 0
        lax.fori_loop(0, SB // (2 * BLK), _pair, 0)

    def _full_sb(b, _):
        off = estart + b * SB
        pltpu.sync_copy(cols.at[pl.ds(off, SB)], colv)
        pltpu.sync_copy(vals.at[pl.ds(off, SB)], valv)
        pltpu.sync_copy(rows.at[pl.ds(off, SB)], rowv)
        _process_superblock()
        return 0
    lax.fori_loop(0, NSB, _full_sb, 0)

    toff = estart + NSB * SB
    pltpu.sync_copy(cols.at[pl.ds(toff, TAIL)], colv.at[pl.ds(0, TAIL)])
    pltpu.sync_copy(vals.at[pl.ds(toff, TAIL)], valv.at[pl.ds(0, TAIL)])
    pltpu.sync_copy(rows.at[pl.ds(toff, TAIL)], rowv.at[pl.ds(0, TAIL)])
    for p in range(TAIL // 16, SB // 16):
        colv[pl.ds(p * 16, 16)] = _Z16I
        valv[pl.ds(p * 16, 16)] = _Z16
        rowv[pl.ds(p * 16, 16)] = _Z16I - 1
    _process_superblock()

    plsc.subcore_barrier()
    pltpu.sync_copy(acc.at[pl.ds(s * 1560, 1560)],
                    out.at[pl.ds(base + s * 1560, 1560)])
    @pl.when(s == 15)
    def _():
        pltpu.sync_copy(acc.at[pl.ds(24960, 40)],
                        out.at[pl.ds(base + 24960, 40)])


_layer = pl.kernel(
    _layer_body,
    out_type=jax.ShapeDtypeStruct((N, D), jnp.float32),
    mesh=_mesh,
    scratch_types=[
        pltpu.VMEM((SB,), jnp.int32),
        pltpu.VMEM((SB,), jnp.int32),
        pltpu.VMEM((SB,), jnp.float32),
        pltpu.VMEM((8, BLK), jnp.int32),
        pltpu.VMEM((BLK, D), jnp.float32),
        pltpu.VMEM((BLK, D), jnp.float32),
        pltpu.VMEM((BLK, D), jnp.float32),
        pltpu.VMEM((12, D), jnp.float32),
        pltpu.VMEM_SHARED((ACC_R, D), jnp.float32),
        pltpu.SemaphoreType.DMA,
        pltpu.SemaphoreType.DMA,
    ],
    compiler_params=pltpu.CompilerParams(use_tc_tiling_on_sc=False),
)


def _final_body(e0, e1, e2, e3, sel, out, idxv, g0, g1, g2, g3, obuf, sem):
    c = lax.axis_index("c")
    s = lax.axis_index("s")
    wid = s * 2 + c
    rpw = (2 * B) // 32

    def _sum_row(e, _):
        for g in range(4):
            sl = pl.ds(g * 16, 16)
            obuf[e, sl] = (g0[e, sl] + g1[e, sl] + g2[e, sl] + g3[e, sl]) * 0.25
        return 0

    def _block(blk, _):
        off = wid * rpw + blk * BLK
        pltpu.sync_copy(sel.at[pl.ds(off, BLK)], idxv)
        pltpu.async_copy(e0.at[idxv], g0, sem).wait()
        pltpu.async_copy(e1.at[idxv], g1, sem).wait()
        pltpu.async_copy(e2.at[idxv], g2, sem).wait()
        pltpu.async_copy(e3.at[idxv], g3, sem).wait()
        lax.fori_loop(0, BLK, _sum_row, 0)
        pltpu.sync_copy(obuf, out.at[pl.ds(off, BLK)])
        return 0
    lax.fori_loop(0, rpw // BLK, _block, 0)


_finalize = pl.kernel(
    _final_body,
    out_type=jax.ShapeDtypeStruct((2 * B, D), jnp.float32),
    mesh=_mesh,
    scratch_types=[
        pltpu.VMEM((BLK,), jnp.int32),
        pltpu.VMEM((BLK, D), jnp.float32),
        pltpu.VMEM((BLK, D), jnp.float32),
        pltpu.VMEM((BLK, D), jnp.float32),
        pltpu.VMEM((BLK, D), jnp.float32),
        pltpu.VMEM((BLK, D), jnp.float32),
        pltpu.SemaphoreType.DMA,
    ],
    compiler_params=pltpu.CompilerParams(use_tc_tiling_on_sc=False),
)


def _pred_body(x_ref, wt_ref, b_ref, o_ref):
    o_ref[...] = jnp.dot(x_ref[...], wt_ref[...],
                         preferred_element_type=jnp.float32) + b_ref[...]


_predict = pl.pallas_call(
    _pred_body,
    out_shape=jax.ShapeDtypeStruct((2 * B, D), jnp.float32),
    grid=(8,),
    in_specs=[
        pl.BlockSpec((2 * B // 8, D), lambda i: (i, 0)),
        pl.BlockSpec((D, D), lambda i: (0, 0)),
        pl.BlockSpec((1, D), lambda i: (0, 0)),
    ],
    out_specs=pl.BlockSpec((2 * B // 8, D), lambda i: (i, 0)),
)


def kernel(user, item, adj_row, adj_col, adj_val, user_emb, item_emb, W, b):
    ego0 = jnp.concatenate([user_emb, item_emb], axis=0)
    sel = jnp.concatenate([user, item + U], axis=0)
    e1 = _layer(ego0, adj_row, adj_col, adj_val)
    e2 = _layer(e1, adj_row, adj_col, adj_val)
    e3 = _layer(e2, adj_row, adj_col, adj_val)
    sm = _finalize(ego0, e1, e2, e3, sel)
    pred = _predict(sm, W.T, b.reshape(1, D))
    return (pred[:B], sm[:B], pred[B:], sm[B:])


# speedup vs baseline: 2.9321x; 1.2555x over previous
import functools

import jax
import jax.numpy as jnp
from jax import lax
from jax.experimental import pallas as pl
from jax.experimental.pallas import tpu as pltpu
from jax.experimental.pallas import tpu_sc as plsc

U = 25000
N = 50000
D = 64
NNZ = 800000
B = 16384

HALF = 25000
ACC_R = 25152
DUM_BASE = 25000
EPT = NNZ // 16
SB = 1024
BLK = 128
NSB = 48
TAIL = EPT - NSB * SB

_mesh = plsc.VectorSubcoreMesh(
    core_axis_name="c", subcore_axis_name="s", num_cores=2, num_subcores=16)

def _layer_body(ego, rows, cols, vals, out, colv, rowv, valv, dstv, gbufa,
                gbufb, sbuf, zbuf, acc, sema, semb):
    c = lax.axis_index("c")
    s = lax.axis_index("s")
    base = c * HALF
    lane = lax.iota(jnp.int32, 16)
    dum = DUM_BASE + s * 8 + (lane & 7)
    _Z16 = jnp.zeros((16,), jnp.float32)
    _Z16I = jnp.zeros((16,), jnp.int32)

    def _zrow(r, _):
        for g in range(4):
            zbuf[r, pl.ds(g * 16, 16)] = _Z16
        return 0
    lax.fori_loop(0, 12, _zrow, 0)

    def _zcopy(k, _):
        pltpu.sync_copy(zbuf, acc.at[pl.ds(s * 1572 + k * 12, 12)])
        return 0
    lax.fori_loop(0, 131, _zcopy, 0)
    plsc.subcore_barrier()

    estart = s * EPT

    def _dst_group(g, _):
        rv = rowv[pl.ds(g * 16, 16)]
        local = rv - base
        msk = (local >= 0) & (local < HALF)
        dv = jnp.where(msk, local, dum)
        j = g // 8
        k = g - j * 8
        dstv[j, pl.ds(k * 16, 16)] = dv
        return 0

    def _process_superblock():
        lax.fori_loop(0, SB // 16, _dst_group, 0)

        npair = SB // (2 * BLK)
        pltpu.async_copy(ego.at[colv.at[pl.ds(0, BLK)]], gbufa, sema)
        pltpu.async_copy(ego.at[colv.at[pl.ds(BLK, BLK)]], gbufb, semb)

        def _pair(q, _):
            ja = q * 2
            jb = q * 2 + 1
            pltpu.make_async_copy(
                ego.at[colv.at[pl.ds(ja * BLK, BLK)]], gbufa, sema).wait()
            for g in range(BLK // 16):
                vv = valv[pl.ds(ja * BLK + g * 16, 16)]
                for l in range(16):
                    v = vv[l]
                    e = g * 16 + l
                    for cg in range(4):
                        sl = pl.ds(cg * 16, 16)
                        sbuf[e, sl] = gbufa[e, sl] * v

            @pl.when(q < npair - 1)
            def _():
                pltpu.async_copy(
                    ego.at[colv.at[pl.ds((ja + 2) * BLK, BLK)]], gbufa, sema)
            pltpu.sync_copy(sbuf, acc.at[dstv.at[ja]], add=True)

            pltpu.make_async_copy(
                ego.at[colv.at[pl.ds(jb * BLK, BLK)]], gbufb, semb).wait()
            for g in range(BLK // 16):
                vv = valv[pl.ds(jb * BLK + g * 16, 16)]
                for l in range(16):
                    v = vv[l]
                    e = g * 16 + l
                    for cg in range(4):
                        sl = pl.ds(cg * 16, 16)
                        sbuf[e, sl] = gbufb[e, sl] * v

            @pl.when(q < npair - 1)
            def _():
                pltpu.async_copy(
                    ego.at[colv.at[pl.ds((jb + 2) * BLK, BLK)]], gbufb, semb)
            pltpu.sync_copy(sbuf, acc.at[dstv.at[jb]], add=True)
            return 0
        lax.fori_loop(0, npair, _pair, 0)

    def _full_sb(b, _):
        off = estart + b * SB
        pltpu.sync_copy(cols.at[pl.ds(off, SB)], colv)
        pltpu.sync_copy(vals.at[pl.ds(off, SB)], valv)
        pltpu.sync_copy(rows.at[pl.ds(off, SB)], rowv)
        _process_superblock()
        return 0
    lax.fori_loop(0, NSB, _full_sb, 0)

    toff = estart + NSB * SB
    pltpu.sync_copy(cols.at[pl.ds(toff, TAIL)], colv.at[pl.ds(0, TAIL)])
    pltpu.sync_copy(vals.at[pl.ds(toff, TAIL)], valv.at[pl.ds(0, TAIL)])
    pltpu.sync_copy(rows.at[pl.ds(toff, TAIL)], rowv.at[pl.ds(0, TAIL)])
    for p in range(TAIL // 16, SB // 16):
        colv[pl.ds(p * 16, 16)] = _Z16I
        valv[pl.ds(p * 16, 16)] = _Z16
        rowv[pl.ds(p * 16, 16)] = _Z16I - 1
    _process_superblock()

    plsc.subcore_barrier()
    pltpu.sync_copy(acc.at[pl.ds(s * 1560, 1560)],
                    out.at[pl.ds(base + s * 1560, 1560)])
    @pl.when(s == 15)
    def _():
        pltpu.sync_copy(acc.at[pl.ds(24960, 40)],
                        out.at[pl.ds(base + 24960, 40)])


_layer = pl.kernel(
    _layer_body,
    out_type=jax.ShapeDtypeStruct((N, D), jnp.float32),
    mesh=_mesh,
    scratch_types=[
        pltpu.VMEM((SB,), jnp.int32),
        pltpu.VMEM((SB,), jnp.int32),
        pltpu.VMEM((SB,), jnp.float32),
        pltpu.VMEM((8, BLK), jnp.int32),
        pltpu.VMEM((BLK, D), jnp.float32),
        pltpu.VMEM((BLK, D), jnp.float32),
        pltpu.VMEM((BLK, D), jnp.float32),
        pltpu.VMEM((12, D), jnp.float32),
        pltpu.VMEM_SHARED((ACC_R, D), jnp.float32),
        pltpu.SemaphoreType.DMA,
        pltpu.SemaphoreType.DMA,
    ],
    compiler_params=pltpu.CompilerParams(use_tc_tiling_on_sc=False),
)


def _final_body(e0, e1, e2, e3, sel, out, idxv, g0, g1, g2, g3, obuf, sem):
    c = lax.axis_index("c")
    s = lax.axis_index("s")
    wid = s * 2 + c
    rpw = (2 * B) // 32

    def _sum_row(e, _):
        for g in range(4):
            sl = pl.ds(g * 16, 16)
            obuf[e, sl] = (g0[e, sl] + g1[e, sl] + g2[e, sl] + g3[e, sl]) * 0.25
        return 0

    def _block(blk, _):
        off = wid * rpw + blk * BLK
        pltpu.sync_copy(sel.at[pl.ds(off, BLK)], idxv)
        pltpu.async_copy(e0.at[idxv], g0, sem).wait()
        pltpu.async_copy(e1.at[idxv], g1, sem).wait()
        pltpu.async_copy(e2.at[idxv], g2, sem).wait()
        pltpu.async_copy(e3.at[idxv], g3, sem).wait()
        lax.fori_loop(0, BLK, _sum_row, 0)
        pltpu.sync_copy(obuf, out.at[pl.ds(off, BLK)])
        return 0
    lax.fori_loop(0, rpw // BLK, _block, 0)


_finalize = pl.kernel(
    _final_body,
    out_type=jax.ShapeDtypeStruct((2 * B, D), jnp.float32),
    mesh=_mesh,
    scratch_types=[
        pltpu.VMEM((BLK,), jnp.int32),
        pltpu.VMEM((BLK, D), jnp.float32),
        pltpu.VMEM((BLK, D), jnp.float32),
        pltpu.VMEM((BLK, D), jnp.float32),
        pltpu.VMEM((BLK, D), jnp.float32),
        pltpu.VMEM((BLK, D), jnp.float32),
        pltpu.SemaphoreType.DMA,
    ],
    compiler_params=pltpu.CompilerParams(use_tc_tiling_on_sc=False),
)


def _pred_body(x_ref, wt_ref, b_ref, o_ref):
    o_ref[...] = jnp.dot(x_ref[...], wt_ref[...],
                         preferred_element_type=jnp.float32) + b_ref[...]


_predict = pl.pallas_call(
    _pred_body,
    out_shape=jax.ShapeDtypeStruct((2 * B, D), jnp.float32),
    grid=(8,),
    in_specs=[
        pl.BlockSpec((2 * B // 8, D), lambda i: (i, 0)),
        pl.BlockSpec((D, D), lambda i: (0, 0)),
        pl.BlockSpec((1, D), lambda i: (0, 0)),
    ],
    out_specs=pl.BlockSpec((2 * B // 8, D), lambda i: (i, 0)),
)


def kernel(user, item, adj_row, adj_col, adj_val, user_emb, item_emb, W, b):
    ego0 = jnp.concatenate([user_emb, item_emb], axis=0)
    sel = jnp.concatenate([user, item + U], axis=0)
    e1 = _layer(ego0, adj_row, adj_col, adj_val)
    e2 = _layer(e1, adj_row, adj_col, adj_val)
    e3 = _layer(e2, adj_row, adj_col, adj_val)
    sm = _finalize(ego0, e1, e2, e3, sel)
    pred = _predict(sm, W.T, b.reshape(1, D))
    return (pred[:B], sm[:B], pred[B:], sm[B:])


# speedup vs baseline: 4.1585x; 1.4183x over previous
import jax
import jax.numpy as jnp
from jax import lax
from jax.experimental import pallas as pl
from jax.experimental.pallas import tpu as pltpu
from jax.experimental.pallas import tpu_sc as plsc

U = 25000
N = 50000
D = 64
NNZ = 800000
B = 16384

HALF = 25000
ACC_R = 25024
EPT = NNZ // 16
SB = 1024
BLK = 128
NSB = 48
TAIL = EPT - NSB * SB
CAP = 53248

_mesh = plsc.VectorSubcoreMesh(
    core_axis_name="c", subcore_axis_name="s", num_cores=2, num_subcores=16)


def _part_body(rows, cols, vals, pcol, pval, pdst, cnts,
               colv, rowv, valv, stc, stv, std, cntstage):
    c = lax.axis_index("c")
    s = lax.axis_index("s")
    w = c * 16 + s
    base = c * HALF
    lane = lax.iota(jnp.int32, 16)
    estart = s * EPT
    rbase = w * CAP
    _Z16 = jnp.zeros((16,), jnp.float32)
    _Z16I = jnp.zeros((16,), jnp.int32)

    def _groups(ngr, carry):
        def _g(g, cr):
            cnt, hoff = cr
            rv = rowv[pl.ds(g * 16, 16)]
            cv = colv[pl.ds(g * 16, 16)]
            vv = valv[pl.ds(g * 16, 16)]
            local = rv - base
            msk = (local >= 0) & (local < HALF)
            pref = plsc.cumsum(jnp.where(msk, 1, 0).astype(jnp.int32))
            pos = cnt + pref - 1
            plsc.store_scatter(stc, [pos], cv, mask=msk)
            plsc.store_scatter(stv, [pos], vv, mask=msk)
            plsc.store_scatter(std, [pos], local, mask=msk)
            cnt2 = cnt + pref[15]
            fl = cnt2 >= 1024

            @pl.when(fl)
            def _():
                fo = pl.multiple_of(rbase + hoff, 1024)
                pltpu.sync_copy(stc.at[pl.ds(0, 1024)],
                                pcol.at[pl.ds(fo, 1024)])
                pltpu.sync_copy(stv.at[pl.ds(0, 1024)],
                                pval.at[pl.ds(fo, 1024)])
                pltpu.sync_copy(std.at[pl.ds(0, 1024)],
                                pdst.at[pl.ds(fo, 1024)])
                stc[pl.ds(0, 16)] = stc[pl.ds(1024, 16)]
                stv[pl.ds(0, 16)] = stv[pl.ds(1024, 16)]
                std[pl.ds(0, 16)] = std[pl.ds(1024, 16)]

            return (jnp.where(fl, cnt2 - 1024, cnt2),
                    jnp.where(fl, hoff + 1024, hoff))
        return lax.fori_loop(0, ngr, _g, carry)

    def _sb(b, cr):
        off = estart + b * SB
        pltpu.sync_copy(cols.at[pl.ds(off, SB)], colv)
        pltpu.sync_copy(vals.at[pl.ds(off, SB)], valv)
        pltpu.sync_copy(rows.at[pl.ds(off, SB)], rowv)
        return _groups(SB // 16, cr)

    cr = lax.fori_loop(0, NSB, _sb, (jnp.int32(0), jnp.int32(0)))

    toff = estart + NSB * SB
    pltpu.sync_copy(cols.at[pl.ds(toff, TAIL)], colv.at[pl.ds(0, TAIL)])
    pltpu.sync_copy(vals.at[pl.ds(toff, TAIL)], valv.at[pl.ds(0, TAIL)])
    pltpu.sync_copy(rows.at[pl.ds(toff, TAIL)], rowv.at[pl.ds(0, TAIL)])
    for p in range(TAIL // 16, SB // 16):
        colv[pl.ds(p * 16, 16)] = _Z16I
        valv[pl.ds(p * 16, 16)] = _Z16
        rowv[pl.ds(p * 16, 16)] = _Z16I - 1
    cnt, hoff = _groups(SB // 16, cr)

    colpad = w * 128 + lane * 8
    dstpad = HALF + (lane & 7)
    for i in range(64):
        pos = cnt + i * 16 + lane
        plsc.store_scatter(stc, [pos], colpad)
        plsc.store_scatter(stv, [pos], _Z16)
        plsc.store_scatter(std, [pos], dstpad)
    n_final = ((cnt + 1023) // 1024) * 1024
    for chunk in range(2):
        o = chunk * 1024
        fo = pl.multiple_of(rbase + hoff + o, 1024)
        pltpu.sync_copy(stc.at[pl.ds(o, 1024)], pcol.at[pl.ds(fo, 1024)])
        pltpu.sync_copy(stv.at[pl.ds(o, 1024)], pval.at[pl.ds(fo, 1024)])
        pltpu.sync_copy(std.at[pl.ds(o, 1024)], pdst.at[pl.ds(fo, 1024)])
    cntstage[pl.ds(0, 16)] = _Z16I + (hoff + n_final)
    pltpu.sync_copy(cntstage, cnts.at[w])


_partition = pl.kernel(
    _part_body,
    out_type=(
        jax.ShapeDtypeStruct((32 * CAP,), jnp.int32),
        jax.ShapeDtypeStruct((32 * CAP,), jnp.float32),
        jax.ShapeDtypeStruct((32 * CAP,), jnp.int32),
        jax.ShapeDtypeStruct((32, 16), jnp.int32),
    ),
    mesh=_mesh,
    scratch_types=[
        pltpu.VMEM((SB,), jnp.int32),
        pltpu.VMEM((SB,), jnp.int32),
        pltpu.VMEM((SB,), jnp.float32),
        pltpu.VMEM((2080,), jnp.int32),
        pltpu.VMEM((2080,), jnp.float32),
        pltpu.VMEM((2080,), jnp.int32),
        pltpu.VMEM((16,), jnp.int32),
    ],
    compiler_params=pltpu.CompilerParams(use_tc_tiling_on_sc=False,
                                         needs_layout_passes=False),
)


def _layer_body(ego, pcol, pval, pdst, cnts, out, colv, valv, dstv,
                gbufa, gbufb, sbuf, zbuf, cntv, acc, sema, semb):
    c = lax.axis_index("c")
    s = lax.axis_index("s")
    w = c * 16 + s
    base = c * HALF
    _Z16 = jnp.zeros((16,), jnp.float32)

    def _zrow(r, _):
        for g in range(4):
            zbuf[r, pl.ds(g * 16, 16)] = _Z16
        return 0
    lax.fori_loop(0, 17, _zrow, 0)

    def _zcopy(k, _):
        pltpu.sync_copy(zbuf, acc.at[pl.ds(s * 1564 + k * 17, 17)])
        return 0
    lax.fori_loop(0, 92, _zcopy, 0)
    plsc.subcore_barrier()

    rbase = w * CAP
    pltpu.sync_copy(cnts.at[w], cntv)
    n = cntv[pl.ds(0, 16)][0]
    nsb = n // SB

    def _sb(b, _):
        off = pl.multiple_of(rbase + b * SB, 1024)
        pltpu.sync_copy(pcol.at[pl.ds(off, SB)], colv)
        pltpu.sync_copy(pval.at[pl.ds(off, SB)], valv)
        for j in range(8):
            pltpu.sync_copy(pdst.at[pl.ds(off + j * BLK, BLK)], dstv.at[j])

        npair = SB // (2 * BLK)
        pltpu.async_copy(ego.at[colv.at[pl.ds(0, BLK)]], gbufa, sema)
        pltpu.async_copy(ego.at[colv.at[pl.ds(BLK, BLK)]], gbufb, semb)

        def _pair(q, _):
            ja = q * 2
            jb = q * 2 + 1
            pltpu.make_async_copy(
                ego.at[colv.at[pl.ds(ja * BLK, BLK)]], gbufa, sema).wait()
            for g in range(BLK // 16):
                vv = valv[pl.ds(ja * BLK + g * 16, 16)]
                for l in range(16):
                    v = vv[l]
                    e = g * 16 + l
                    for cg in range(4):
                        sl = pl.ds(cg * 16, 16)
                        sbuf[e, sl] = gbufa[e, sl] * v

            @pl.when(q < npair - 1)
            def _():
                pltpu.async_copy(
                    ego.at[colv.at[pl.ds((ja + 2) * BLK, BLK)]], gbufa, sema)
            pltpu.sync_copy(sbuf, acc.at[dstv.at[ja]], add=True)

            pltpu.make_async_copy(
                ego.at[colv.at[pl.ds(jb * BLK, BLK)]], gbufb, semb).wait()
            for g in range(BLK // 16):
                vv = valv[pl.ds(jb * BLK + g * 16, 16)]
                for l in range(16):
                    v = vv[l]
                    e = g * 16 + l
                    for cg in range(4):
                        sl = pl.ds(cg * 16, 16)
                        sbuf[e, sl] = gbufb[e, sl] * v

            @pl.when(q < npair - 1)
            def _():
                pltpu.async_copy(
                    ego.at[colv.at[pl.ds((jb + 2) * BLK, BLK)]], gbufb, semb)
            pltpu.sync_copy(sbuf, acc.at[dstv.at[jb]], add=True)
            return 0
        lax.fori_loop(0, npair, _pair, 0)
        return 0
    lax.fori_loop(0, nsb, _sb, 0)

    plsc.subcore_barrier()
    pltpu.sync_copy(acc.at[pl.ds(s * 1560, 1560)],
                    out.at[pl.ds(base + s * 1560, 1560)])
    @pl.when(s == 15)
    def _():
        pltpu.sync_copy(acc.at[pl.ds(24960, 40)],
                        out.at[pl.ds(base + 24960, 40)])


_layer = pl.kernel(
    _layer_body,
    out_type=jax.ShapeDtypeStruct((N, D), jnp.float32),
    mesh=_mesh,
    scratch_types=[
        pltpu.VMEM((SB,), jnp.int32),
        pltpu.VMEM((SB,), jnp.float32),
        pltpu.VMEM((8, BLK), jnp.int32),
        pltpu.VMEM((BLK, D), jnp.float32),
        pltpu.VMEM((BLK, D), jnp.float32),
        pltpu.VMEM((BLK, D), jnp.float32),
        pltpu.VMEM((17, D), jnp.float32),
        pltpu.VMEM((16,), jnp.int32),
        pltpu.VMEM_SHARED((ACC_R, D), jnp.float32),
        pltpu.SemaphoreType.DMA,
        pltpu.SemaphoreType.DMA,
    ],
    compiler_params=pltpu.CompilerParams(use_tc_tiling_on_sc=False),
)


def _final_body(e0, e1, e2, e3, sel, out, idxv, g0, g1, g2, g3, obuf, sem):
    c = lax.axis_index("c")
    s = lax.axis_index("s")
    wid = s * 2 + c
    rpw = (2 * B) // 32

    def _sum_row(e, _):
        for g in range(4):
            sl = pl.ds(g * 16, 16)
            obuf[e, sl] = (g0[e, sl] + g1[e, sl] + g2[e, sl] + g3[e, sl]) * 0.25
        return 0

    def _block(blk, _):
        off = wid * rpw + blk * BLK
        pltpu.sync_copy(sel.at[pl.ds(off, BLK)], idxv)
        pltpu.async_copy(e0.at[idxv], g0, sem).wait()
        pltpu.async_copy(e1.at[idxv], g1, sem).wait()
        pltpu.async_copy(e2.at[idxv], g2, sem).wait()
        pltpu.async_copy(e3.at[idxv], g3, sem).wait()
        lax.fori_loop(0, BLK, _sum_row, 0)
        pltpu.sync_copy(obuf, out.at[pl.ds(off, BLK)])
        return 0
    lax.fori_loop(0, rpw // BLK, _block, 0)


_finalize = pl.kernel(
    _final_body,
    out_type=jax.ShapeDtypeStruct((2 * B, D), jnp.float32),
    mesh=_mesh,
    scratch_types=[
        pltpu.VMEM((BLK,), jnp.int32),
        pltpu.VMEM((BLK, D), jnp.float32),
        pltpu.VMEM((BLK, D), jnp.float32),
        pltpu.VMEM((BLK, D), jnp.float32),
        pltpu.VMEM((BLK, D), jnp.float32),
        pltpu.VMEM((BLK, D), jnp.float32),
        pltpu.SemaphoreType.DMA,
    ],
    compiler_params=pltpu.CompilerParams(use_tc_tiling_on_sc=False),
)


def _pred_body(x_ref, wt_ref, b_ref, o_ref):
    o_ref[...] = jnp.dot(x_ref[...], wt_ref[...],
                         preferred_element_type=jnp.float32) + b_ref[...]


_predict = pl.pallas_call(
    _pred_body,
    out_shape=jax.ShapeDtypeStruct((2 * B, D), jnp.float32),
    grid=(8,),
    in_specs=[
        pl.BlockSpec((2 * B // 8, D), lambda i: (i, 0)),
        pl.BlockSpec((D, D), lambda i: (0, 0)),
        pl.BlockSpec((1, D), lambda i: (0, 0)),
    ],
    out_specs=pl.BlockSpec((2 * B // 8, D), lambda i: (i, 0)),
)


def kernel(user, item, adj_row, adj_col, adj_val, user_emb, item_emb, W, b):
    ego0 = jnp.concatenate([user_emb, item_emb], axis=0)
    sel = jnp.concatenate([user, item + U], axis=0)
    pcol, pval, pdst, cnts = _partition(adj_row, adj_col, adj_val)
    e1 = _layer(ego0, pcol, pval, pdst, cnts)
    e2 = _layer(e1, pcol, pval, pdst, cnts)
    e3 = _layer(e2, pcol, pval, pdst, cnts)
    sm = _finalize(ego0, e1, e2, e3, sel)
    pred = _predict(sm, W.T, b.reshape(1, D))
    return (pred[:B], sm[:B], pred[B:], sm[B:])


# speedup vs baseline: 5.0386x; 1.2116x over previous
import jax
import jax.numpy as jnp
from jax import lax
from jax.experimental import pallas as pl
from jax.experimental.pallas import tpu as pltpu
from jax.experimental.pallas import tpu_sc as plsc

U = 25000
N = 50000
D = 64
NNZ = 800000
B = 16384

HALF = 25000
ACC_R = 25024
EPT = NNZ // 16
SB = 1024
BLK = 128
NSB = 48
TAIL = EPT - NSB * SB
CAP = 53248

_mesh = plsc.VectorSubcoreMesh(
    core_axis_name="c", subcore_axis_name="s", num_cores=2, num_subcores=16)


def _part_body(rows, cols, vals, pcol, pval, pdst, cnts,
               colv, rowv, valv, stc, stv, std, cntstage):
    c = lax.axis_index("c")
    s = lax.axis_index("s")
    w = c * 16 + s
    base = c * HALF
    lane = lax.iota(jnp.int32, 16)
    estart = s * EPT
    rbase = w * CAP
    _Z16 = jnp.zeros((16,), jnp.float32)
    _Z16I = jnp.zeros((16,), jnp.int32)

    def _groups(ngr, carry):
        def _g(g, cr):
            cnt, hoff = cr
            rv = rowv[pl.ds(g * 16, 16)]
            cv = colv[pl.ds(g * 16, 16)]
            vv = valv[pl.ds(g * 16, 16)]
            local = rv - base
            msk = (local >= 0) & (local < HALF)
            pref = plsc.cumsum(jnp.where(msk, 1, 0).astype(jnp.int32))
            pos = cnt + pref - 1
            plsc.store_scatter(stc, [pos], cv, mask=msk)
            plsc.store_scatter(stv, [pos], vv, mask=msk)
            plsc.store_scatter(std, [pos >> 7, pos & 127], local, mask=msk)
            cnt2 = cnt + pref[15]
            fl = cnt2 >= 1024

            @pl.when(fl)
            def _():
                fo = pl.multiple_of(rbase + hoff, 1024)
                pltpu.sync_copy(stc.at[pl.ds(0, 1024)],
                                pcol.at[pl.ds(fo, 1024)])
                pltpu.sync_copy(stv.at[pl.ds(0, 1024)],
                                pval.at[pl.ds(fo, 1024)])
                pltpu.sync_copy(std.at[pl.ds(0, 8)],
                                pdst.at[pl.ds(fo >> 7, 8)])
                stc[pl.ds(0, 16)] = stc[pl.ds(1024, 16)]
                stv[pl.ds(0, 16)] = stv[pl.ds(1024, 16)]
                std[0, pl.ds(0, 16)] = std[8, pl.ds(0, 16)]

            return (jnp.where(fl, cnt2 - 1024, cnt2),
                    jnp.where(fl, hoff + 1024, hoff))
        return lax.fori_loop(0, ngr, _g, carry)

    def _sb(b, cr):
        off = estart + b * SB
        pltpu.sync_copy(cols.at[pl.ds(off, SB)], colv)
        pltpu.sync_copy(vals.at[pl.ds(off, SB)], valv)
        pltpu.sync_copy(rows.at[pl.ds(off, SB)], rowv)
        return _groups(SB // 16, cr)

    cr = lax.fori_loop(0, NSB, _sb, (jnp.int32(0), jnp.int32(0)))

    toff = estart + NSB * SB
    pltpu.sync_copy(cols.at[pl.ds(toff, TAIL)], colv.at[pl.ds(0, TAIL)])
    pltpu.sync_copy(vals.at[pl.ds(toff, TAIL)], valv.at[pl.ds(0, TAIL)])
    pltpu.sync_copy(rows.at[pl.ds(toff, TAIL)], rowv.at[pl.ds(0, TAIL)])
    for p in range(TAIL // 16, SB // 16):
        colv[pl.ds(p * 16, 16)] = _Z16I
        valv[pl.ds(p * 16, 16)] = _Z16
        rowv[pl.ds(p * 16, 16)] = _Z16I - 1
    cnt, hoff = _groups(SB // 16, cr)

    colpad = w * 128 + lane * 8
    dstpad = HALF + (lane & 7)
    for i in range(64):
        pos = cnt + i * 16 + lane
        plsc.store_scatter(stc, [pos], colpad)
        plsc.store_scatter(stv, [pos], _Z16)
        plsc.store_scatter(std, [pos >> 7, pos & 127], dstpad)
    n_final = ((cnt + 1023) // 1024) * 1024
    for chunk in range(2):
        o = chunk * 1024
        fo = pl.multiple_of(rbase + hoff + o, 1024)
        pltpu.sync_copy(stc.at[pl.ds(o, 1024)], pcol.at[pl.ds(fo, 1024)])
        pltpu.sync_copy(stv.at[pl.ds(o, 1024)], pval.at[pl.ds(fo, 1024)])
        pltpu.sync_copy(std.at[pl.ds(chunk * 8, 8)],
                        pdst.at[pl.ds(fo >> 7, 8)])
    cntstage[pl.ds(0, 16)] = _Z16I + (hoff + n_final)
    pltpu.sync_copy(cntstage, cnts.at[w])


_partition = pl.kernel(
    _part_body,
    out_type=(
        jax.ShapeDtypeStruct((32 * CAP,), jnp.int32),
        jax.ShapeDtypeStruct((32 * CAP,), jnp.float32),
        jax.ShapeDtypeStruct((32 * CAP // 128, 128), jnp.int32),
        jax.ShapeDtypeStruct((32, 16), jnp.int32),
    ),
    mesh=_mesh,
    scratch_types=[
        pltpu.VMEM((SB,), jnp.int32),
        pltpu.VMEM((SB,), jnp.int32),
        pltpu.VMEM((SB,), jnp.float32),
        pltpu.VMEM((2080,), jnp.int32),
        pltpu.VMEM((2080,), jnp.float32),
        pltpu.VMEM((17, 128), jnp.int32),
        pltpu.VMEM((16,), jnp.int32),
    ],
    compiler_params=pltpu.CompilerParams(use_tc_tiling_on_sc=False,
                                         needs_layout_passes=False),
)


def _layer_body(ego, pcol, pval, pdst, cnts, out, colv, valv, dstv,
                gbufa, gbufb, sbuf, zbuf, cntv, acc, sema, semb):
    c = lax.axis_index("c")
    s = lax.axis_index("s")
    w = c * 16 + s
    base = c * HALF
    _Z16 = jnp.zeros((16,), jnp.float32)

    def _zrow(r, _):
        for g in range(4):
            zbuf[r, pl.ds(g * 16, 16)] = _Z16
        return 0
    lax.fori_loop(0, 17, _zrow, 0)

    def _zcopy(k, _):
        pltpu.sync_copy(zbuf, acc.at[pl.ds(s * 1564 + k * 17, 17)])
        return 0
    lax.fori_loop(0, 92, _zcopy, 0)
    plsc.subcore_barrier()

    rbase = w * CAP
    pltpu.sync_copy(cnts.at[w], cntv)
    n = cntv[pl.ds(0, 16)][0]
    nsb = n // SB

    def _sb(b, _):
        off = pl.multiple_of(rbase + b * SB, 1024)
        pltpu.sync_copy(pcol.at[pl.ds(off, SB)], colv)
        pltpu.sync_copy(pval.at[pl.ds(off, SB)], valv)
        pltpu.sync_copy(pdst.at[pl.ds(off >> 7, 8)], dstv)

        npair = SB // (2 * BLK)
        pltpu.async_copy(ego.at[colv.at[pl.ds(0, BLK)]], gbufa, sema)
        pltpu.async_copy(ego.at[colv.at[pl.ds(BLK, BLK)]], gbufb, semb)

        def _pair(q, _):
            ja = q * 2
            jb = q * 2 + 1
            pltpu.make_async_copy(
                ego.at[colv.at[pl.ds(ja * BLK, BLK)]], gbufa, sema).wait()
            for g in range(BLK // 16):
                vv = valv[pl.ds(ja * BLK + g * 16, 16)]
                for l in range(16):
                    v = vv[l]
                    e = g * 16 + l
                    for cg in range(4):
                        sl = pl.ds(cg * 16, 16)
                        sbuf[e, sl] = gbufa[e, sl] * v

            @pl.when(q < npair - 1)
            def _():
                pltpu.async_copy(
                    ego.at[colv.at[pl.ds((ja + 2) * BLK, BLK)]], gbufa, sema)
            pltpu.sync_copy(sbuf, acc.at[dstv.at[ja]], add=True)

            pltpu.make_async_copy(
                ego.at[colv.at[pl.ds(jb * BLK, BLK)]], gbufb, semb).wait()
            for g in range(BLK // 16):
                vv = valv[pl.ds(jb * BLK + g * 16, 16)]
                for l in range(16):
                    v = vv[l]
                    e = g * 16 + l
                    for cg in range(4):
                        sl = pl.ds(cg * 16, 16)
                        sbuf[e, sl] = gbufb[e, sl] * v

            @pl.when(q < npair - 1)
            def _():
                pltpu.async_copy(
                    ego.at[colv.at[pl.ds((jb + 2) * BLK, BLK)]], gbufb, semb)
            pltpu.sync_copy(sbuf, acc.at[dstv.at[jb]], add=True)
            return 0
        lax.fori_loop(0, npair, _pair, 0)
        return 0
    lax.fori_loop(0, nsb, _sb, 0)

    plsc.subcore_barrier()
    pltpu.sync_copy(acc.at[pl.ds(s * 1560, 1560)],
                    out.at[pl.ds(base + s * 1560, 1560)])
    @pl.when(s == 15)
    def _():
        pltpu.sync_copy(acc.at[pl.ds(24960, 40)],
                        out.at[pl.ds(base + 24960, 40)])


_layer = pl.kernel(
    _layer_body,
    out_type=jax.ShapeDtypeStruct((N, D), jnp.float32),
    mesh=_mesh,
    scratch_types=[
        pltpu.VMEM((SB,), jnp.int32),
        pltpu.VMEM((SB,), jnp.float32),
        pltpu.VMEM((8, BLK), jnp.int32),
        pltpu.VMEM((BLK, D), jnp.float32),
        pltpu.VMEM((BLK, D), jnp.float32),
        pltpu.VMEM((BLK, D), jnp.float32),
        pltpu.VMEM((17, D), jnp.float32),
        pltpu.VMEM((16,), jnp.int32),
        pltpu.VMEM_SHARED((ACC_R, D), jnp.float32),
        pltpu.SemaphoreType.DMA,
        pltpu.SemaphoreType.DMA,
    ],
    compiler_params=pltpu.CompilerParams(use_tc_tiling_on_sc=False),
)


def _final_body(e0, e1, e2, e3, sel, out, idxv, g0, g1, g2, g3, obuf, sem):
    c = lax.axis_index("c")
    s = lax.axis_index("s")
    wid = s * 2 + c
    rpw = (2 * B) // 32

    def _sum_row(e, _):
        for g in range(4):
            sl = pl.ds(g * 16, 16)
            obuf[e, sl] = (g0[e, sl] + g1[e, sl] + g2[e, sl] + g3[e, sl]) * 0.25
        return 0

    def _block(blk, _):
        off = wid * rpw + blk * BLK
        pltpu.sync_copy(sel.at[pl.ds(off, BLK)], idxv)
        pltpu.async_copy(e0.at[idxv], g0, sem).wait()
        pltpu.async_copy(e1.at[idxv], g1, sem).wait()
        pltpu.async_copy(e2.at[idxv], g2, sem).wait()
        pltpu.async_copy(e3.at[idxv], g3, sem).wait()
        lax.fori_loop(0, BLK, _sum_row, 0)
        pltpu.sync_copy(obuf, out.at[pl.ds(off, BLK)])
        return 0
    lax.fori_loop(0, rpw // BLK, _block, 0)


_finalize = pl.kernel(
    _final_body,
    out_type=jax.ShapeDtypeStruct((2 * B, D), jnp.float32),
    mesh=_mesh,
    scratch_types=[
        pltpu.VMEM((BLK,), jnp.int32),
        pltpu.VMEM((BLK, D), jnp.float32),
        pltpu.VMEM((BLK, D), jnp.float32),
        pltpu.VMEM((BLK, D), jnp.float32),
        pltpu.VMEM((BLK, D), jnp.float32),
        pltpu.VMEM((BLK, D), jnp.float32),
        pltpu.SemaphoreType.DMA,
    ],
    compiler_params=pltpu.CompilerParams(use_tc_tiling_on_sc=False),
)


def _pred_body(x_ref, wt_ref, b_ref, o_ref):
    o_ref[...] = jnp.dot(x_ref[...], wt_ref[...],
                         preferred_element_type=jnp.float32) + b_ref[...]


_predict = pl.pallas_call(
    _pred_body,
    out_shape=jax.ShapeDtypeStruct((2 * B, D), jnp.float32),
    grid=(8,),
    in_specs=[
        pl.BlockSpec((2 * B // 8, D), lambda i: (i, 0)),
        pl.BlockSpec((D, D), lambda i: (0, 0)),
        pl.BlockSpec((1, D), lambda i: (0, 0)),
    ],
    out_specs=pl.BlockSpec((2 * B // 8, D), lambda i: (i, 0)),
)


def kernel(user, item, adj_row, adj_col, adj_val, user_emb, item_emb, W, b):
    ego0 = jnp.concatenate([user_emb, item_emb], axis=0)
    sel = jnp.concatenate([user, item + U], axis=0)
    pcol, pval, pdst, cnts = _partition(adj_row, adj_col, adj_val)
    e1 = _layer(ego0, pcol, pval, pdst, cnts)
    e2 = _layer(e1, pcol, pval, pdst, cnts)
    e3 = _layer(e2, pcol, pval, pdst, cnts)
    sm = _finalize(ego0, e1, e2, e3, sel)
    pred = _predict(sm, W.T, b.reshape(1, D))
    return (pred[:B], sm[:B], pred[B:], sm[B:])


# speedup vs baseline: 5.5153x; 1.0946x over previous
import jax
import jax.numpy as jnp
from jax import lax
from jax.experimental import pallas as pl
from jax.experimental.pallas import tpu as pltpu
from jax.experimental.pallas import tpu_sc as plsc

U = 25000
N = 50000
D = 64
NNZ = 800000
B = 16384

HALF = 25000
ACC_R = 25024
EPT = NNZ // 16
SB = 1024
BLK = 128
NSB = 48
TAIL = EPT - NSB * SB
CAP = 53248

_mesh = plsc.VectorSubcoreMesh(
    core_axis_name="c", subcore_axis_name="s", num_cores=2, num_subcores=16)


def _part_body(rows, cols, vals, pcol, pval, pdst, cnts,
               colv, rowv, valv, stc, stv, std, cntstage):
    c = lax.axis_index("c")
    s = lax.axis_index("s")
    w = c * 16 + s
    base = c * HALF
    lane = lax.iota(jnp.int32, 16)
    estart = s * EPT
    rbase = w * CAP
    _Z16 = jnp.zeros((16,), jnp.float32)
    _Z16I = jnp.zeros((16,), jnp.int32)

    def _groups(ngr, carry):
        def _g(g, cr):
            cnt, hoff = cr
            rv = rowv[pl.ds(g * 16, 16)]
            cv = colv[pl.ds(g * 16, 16)]
            vv = valv[pl.ds(g * 16, 16)]
            local = rv - base
            msk = (local >= 0) & (local < HALF)
            pref = plsc.cumsum(jnp.where(msk, 1, 0).astype(jnp.int32))
            pos = cnt + pref - 1
            plsc.store_scatter(stc, [pos], cv, mask=msk)
            plsc.store_scatter(stv, [pos], vv, mask=msk)
            plsc.store_scatter(std, [pos >> 7, pos & 127], local, mask=msk)
            cnt2 = cnt + pref[15]
            fl = cnt2 >= 1024

            @pl.when(fl)
            def _():
                fo = pl.multiple_of(rbase + hoff, 1024)
                pltpu.sync_copy(stc.at[pl.ds(0, 1024)],
                                pcol.at[pl.ds(fo, 1024)])
                pltpu.sync_copy(stv.at[pl.ds(0, 1024)],
                                pval.at[pl.ds(fo, 1024)])
                pltpu.sync_copy(std.at[pl.ds(0, 8)],
                                pdst.at[pl.ds(fo >> 7, 8)])
                stc[pl.ds(0, 16)] = stc[pl.ds(1024, 16)]
                stv[pl.ds(0, 16)] = stv[pl.ds(1024, 16)]
                std[0, pl.ds(0, 16)] = std[8, pl.ds(0, 16)]

            return (jnp.where(fl, cnt2 - 1024, cnt2),
                    jnp.where(fl, hoff + 1024, hoff))
        return lax.fori_loop(0, ngr, _g, carry)

    def _sb(b, cr):
        off = estart + b * SB
        pltpu.sync_copy(cols.at[pl.ds(off, SB)], colv)
        pltpu.sync_copy(vals.at[pl.ds(off, SB)], valv)
        pltpu.sync_copy(rows.at[pl.ds(off, SB)], rowv)
        return _groups(SB // 16, cr)

    cr = lax.fori_loop(0, NSB, _sb, (jnp.int32(0), jnp.int32(0)))

    toff = estart + NSB * SB
    pltpu.sync_copy(cols.at[pl.ds(toff, TAIL)], colv.at[pl.ds(0, TAIL)])
    pltpu.sync_copy(vals.at[pl.ds(toff, TAIL)], valv.at[pl.ds(0, TAIL)])
    pltpu.sync_copy(rows.at[pl.ds(toff, TAIL)], rowv.at[pl.ds(0, TAIL)])
    for p in range(TAIL // 16, SB // 16):
        colv[pl.ds(p * 16, 16)] = _Z16I
        valv[pl.ds(p * 16, 16)] = _Z16
        rowv[pl.ds(p * 16, 16)] = _Z16I - 1
    cnt, hoff = _groups(SB // 16, cr)

    colpad = w * 128 + lane * 8
    dstpad = HALF + (lane & 7)
    for i in range(64):
        pos = cnt + i * 16 + lane
        plsc.store_scatter(stc, [pos], colpad)
        plsc.store_scatter(stv, [pos], _Z16)
        plsc.store_scatter(std, [pos >> 7, pos & 127], dstpad)
    n_final = ((cnt + 1023) // 1024) * 1024
    for chunk in range(2):
        o = chunk * 1024
        fo = pl.multiple_of(rbase + hoff + o, 1024)
        pltpu.sync_copy(stc.at[pl.ds(o, 1024)], pcol.at[pl.ds(fo, 1024)])
        pltpu.sync_copy(stv.at[pl.ds(o, 1024)], pval.at[pl.ds(fo, 1024)])
        pltpu.sync_copy(std.at[pl.ds(chunk * 8, 8)],
                        pdst.at[pl.ds(fo >> 7, 8)])
    cntstage[pl.ds(0, 16)] = _Z16I + (hoff + n_final)
    pltpu.sync_copy(cntstage, cnts.at[w])


_partition = pl.kernel(
    _part_body,
    out_type=(
        jax.ShapeDtypeStruct((32 * CAP,), jnp.int32),
        jax.ShapeDtypeStruct((32 * CAP,), jnp.float32),
        jax.ShapeDtypeStruct((32 * CAP // 128, 128), jnp.int32),
        jax.ShapeDtypeStruct((32, 16), jnp.int32),
    ),
    mesh=_mesh,
    scratch_types=[
        pltpu.VMEM((SB,), jnp.int32),
        pltpu.VMEM((SB,), jnp.int32),
        pltpu.VMEM((SB,), jnp.float32),
        pltpu.VMEM((2080,), jnp.int32),
        pltpu.VMEM((2080,), jnp.float32),
        pltpu.VMEM((17, 128), jnp.int32),
        pltpu.VMEM((16,), jnp.int32),
    ],
    compiler_params=pltpu.CompilerParams(use_tc_tiling_on_sc=False,
                                         needs_layout_passes=False),
)


def _layer_body(ego, pcol, pval, pdst, cnts, out, colv, valv, dstv,
                gbufa, gbufb, sbuf, zbuf, cntv, acc, sema, semb):
    c = lax.axis_index("c")
    s = lax.axis_index("s")
    w = c * 16 + s
    base = c * HALF
    _Z16 = jnp.zeros((16,), jnp.float32)

    def _zrow(r, _):
        for g in range(4):
            zbuf[r, pl.ds(g * 16, 16)] = _Z16
        return 0
    lax.fori_loop(0, 17, _zrow, 0)

    def _zcopy(k, _):
        pltpu.sync_copy(zbuf, acc.at[pl.ds(s * 1564 + k * 17, 17)])
        return 0
    lax.fori_loop(0, 92, _zcopy, 0)
    plsc.subcore_barrier()

    rbase = w * CAP
    pltpu.sync_copy(cnts.at[w], cntv)
    n = cntv[pl.ds(0, 16)][0]
    nsb = n // SB

    def _sb(b, _):
        off = pl.multiple_of(rbase + b * SB, 1024)
        dl = [pltpu.async_copy(pcol.at[pl.ds(off, SB)], colv, sema),
              pltpu.async_copy(pval.at[pl.ds(off, SB)], valv, sema),
              pltpu.async_copy(pdst.at[pl.ds(off >> 7, 8)], dstv, sema)]
        for d in dl:
            d.wait()

        npair = SB // (2 * BLK)
        pltpu.async_copy(ego.at[colv.at[pl.ds(0, BLK)]], gbufa, sema)
        pltpu.async_copy(ego.at[colv.at[pl.ds(BLK, BLK)]], gbufb, semb)

        def _pair(q, _):
            ja = q * 2
            jb = q * 2 + 1
            pltpu.make_async_copy(
                ego.at[colv.at[pl.ds(ja * BLK, BLK)]], gbufa, sema).wait()
            for g in range(BLK // 16):
                vv = valv[pl.ds(ja * BLK + g * 16, 16)]
                for l in range(16):
                    v = vv[l]
                    e = g * 16 + l
                    for cg in range(4):
                        sl = pl.ds(cg * 16, 16)
                        sbuf[e, sl] = gbufa[e, sl] * v

            @pl.when(q < npair - 1)
            def _():
                pltpu.async_copy(
                    ego.at[colv.at[pl.ds((ja + 2) * BLK, BLK)]], gbufa, sema)
            pltpu.sync_copy(sbuf, acc.at[dstv.at[ja]], add=True)

            pltpu.make_async_copy(
                ego.at[colv.at[pl.ds(jb * BLK, BLK)]], gbufb, semb).wait()
            for g in range(BLK // 16):
                vv = valv[pl.ds(jb * BLK + g * 16, 16)]
                for l in range(16):
                    v = vv[l]
                    e = g * 16 + l
                    for cg in range(4):
                        sl = pl.ds(cg * 16, 16)
                        sbuf[e, sl] = gbufb[e, sl] * v

            @pl.when(q < npair - 1)
            def _():
                pltpu.async_copy(
                    ego.at[colv.at[pl.ds((jb + 2) * BLK, BLK)]], gbufb, semb)
            pltpu.sync_copy(sbuf, acc.at[dstv.at[jb]], add=True)
            return 0
        lax.fori_loop(0, npair, _pair, 0)
        return 0
    lax.fori_loop(0, nsb, _sb, 0)

    plsc.subcore_barrier()
    pltpu.sync_copy(acc.at[pl.ds(s * 1560, 1560)],
                    out.at[pl.ds(base + s * 1560, 1560)])
    @pl.when(s == 15)
    def _():
        pltpu.sync_copy(acc.at[pl.ds(24960, 40)],
                        out.at[pl.ds(base + 24960, 40)])


_layer = pl.kernel(
    _layer_body,
    out_type=jax.ShapeDtypeStruct((N, D), jnp.float32),
    mesh=_mesh,
    scratch_types=[
        pltpu.VMEM((SB,), jnp.int32),
        pltpu.VMEM((SB,), jnp.float32),
        pltpu.VMEM((8, BLK), jnp.int32),
        pltpu.VMEM((BLK, D), jnp.float32),
        pltpu.VMEM((BLK, D), jnp.float32),
        pltpu.VMEM((BLK, D), jnp.float32),
        pltpu.VMEM((17, D), jnp.float32),
        pltpu.VMEM((16,), jnp.int32),
        pltpu.VMEM_SHARED((ACC_R, D), jnp.float32),
        pltpu.SemaphoreType.DMA,
        pltpu.SemaphoreType.DMA,
    ],
    compiler_params=pltpu.CompilerParams(use_tc_tiling_on_sc=False),
)


def _final_body(e0, e1, e2, e3, sel, out, idxv, g0, g1, g2, g3, obuf, sem):
    c = lax.axis_index("c")
    s = lax.axis_index("s")
    wid = s * 2 + c
    rpw = (2 * B) // 32

    def _sum_row(e, _):
        for g in range(4):
            sl = pl.ds(g * 16, 16)
            obuf[e, sl] = (g0[e, sl] + g1[e, sl] + g2[e, sl] + g3[e, sl]) * 0.25
        return 0

    def _block(blk, _):
        off = wid * rpw + blk * BLK
        pltpu.sync_copy(sel.at[pl.ds(off, BLK)], idxv)
        dl = [pltpu.async_copy(e0.at[idxv], g0, sem),
              pltpu.async_copy(e1.at[idxv], g1, sem),
              pltpu.async_copy(e2.at[idxv], g2, sem),
              pltpu.async_copy(e3.at[idxv], g3, sem)]
        for d in dl:
            d.wait()
        lax.fori_loop(0, BLK, _sum_row, 0)
        pltpu.sync_copy(obuf, out.at[pl.ds(off, BLK)])
        return 0
    lax.fori_loop(0, rpw // BLK, _block, 0)


_finalize = pl.kernel(
    _final_body,
    out_type=jax.ShapeDtypeStruct((2 * B, D), jnp.float32),
    mesh=_mesh,
    scratch_types=[
        pltpu.VMEM((BLK,), jnp.int32),
        pltpu.VMEM((BLK, D), jnp.float32),
        pltpu.VMEM((BLK, D), jnp.float32),
        pltpu.VMEM((BLK, D), jnp.float32),
        pltpu.VMEM((BLK, D), jnp.float32),
        pltpu.VMEM((BLK, D), jnp.float32),
        pltpu.SemaphoreType.DMA,
    ],
    compiler_params=pltpu.CompilerParams(use_tc_tiling_on_sc=False),
)


def _pred_body(x_ref, wt_ref, b_ref, o_ref):
    o_ref[...] = jnp.dot(x_ref[...], wt_ref[...],
                         preferred_element_type=jnp.float32) + b_ref[...]


_predict = pl.pallas_call(
    _pred_body,
    out_shape=jax.ShapeDtypeStruct((2 * B, D), jnp.float32),
    grid=(8,),
    in_specs=[
        pl.BlockSpec((2 * B // 8, D), lambda i: (i, 0)),
        pl.BlockSpec((D, D), lambda i: (0, 0)),
        pl.BlockSpec((1, D), lambda i: (0, 0)),
    ],
    out_specs=pl.BlockSpec((2 * B // 8, D), lambda i: (i, 0)),
)


def kernel(user, item, adj_row, adj_col, adj_val, user_emb, item_emb, W, b):
    ego0 = jnp.concatenate([user_emb, item_emb], axis=0)
    sel = jnp.concatenate([user, item + U], axis=0)
    pcol, pval, pdst, cnts = _partition(adj_row, adj_col, adj_val)
    e1 = _layer(ego0, pcol, pval, pdst, cnts)
    e2 = _layer(e1, pcol, pval, pdst, cnts)
    e3 = _layer(e2, pcol, pval, pdst, cnts)
    sm = _finalize(ego0, e1, e2, e3, sel)
    pred = _predict(sm, W.T, b.reshape(1, D))
    return (pred[:B], sm[:B], pred[B:], sm[B:])
